# baseline (speedup 1.0000x reference)
import functools

import jax
import jax.numpy as jnp
from jax.experimental import pallas as pl
from jax.experimental.pallas import tpu as pltpu


def _concat_body(n, g, k, w, r_ref, g_ref, o_ref):
    i = pl.program_id(0)
    o_ref[...] = r_ref[...]

    @pl.when(i == k - 1)
    def _():
        off = n - (k - 1) * w
        o_ref[:, :, off:off + g] = g_ref[...]


def kernel(rois, gt_boxes):
    B, N, C = rois.shape
    _, G, _ = gt_boxes.shape
    r_t = jnp.transpose(rois, (0, 2, 1))
    g_t = jnp.transpose(gt_boxes, (0, 2, 1))
    K = 2
    W = -(-(N + G) // (K * 128)) * 128
    body = functools.partial(_concat_body, N, G, K, W)
    out_t = pl.pallas_call(
        body,
        grid=(K,),
        in_specs=[
            pl.BlockSpec((B, C, W), lambda i: (0, 0, i)),
            pl.BlockSpec((B, C, G), lambda i: (0, 0, 0)),
        ],
        out_specs=pl.BlockSpec((B, C, W), lambda i: (0, 0, i)),
        out_shape=jax.ShapeDtypeStruct((B, C, N + G), rois.dtype),
    )(r_t, g_t)
    return jnp.transpose(out_t, (0, 2, 1))
